# baseline (speedup 1.0000x reference)
import jax
import jax.numpy as jnp
from jax import lax
from jax.experimental import pallas as pl
from jax.experimental.pallas import tpu as pltpu

_B, _L, _D = 1024, 50, 1000
_W = 8


def _body(ladj_ref, pred_ref, tgt_ref, out_ref, acc_ref):
    b = pl.program_id(0)

    @pl.when(b == 0)
    def _init():
        acc_ref[...] = jnp.zeros_like(acc_ref)

    sub = ladj_ref[b] % _W
    p = pred_ref[0]
    t = tgt_ref[0].astype(jnp.float32)
    d = p - t
    rowmask = lax.broadcasted_iota(jnp.int32, (_W, _D), 0) == sub
    acc_ref[...] += jnp.where(rowmask, d * d, 0.0)

    @pl.when(b == _B - 1)
    def _fin():
        out_ref[0, 0] = jnp.sum(acc_ref[...]) * (1.0 / (_B * _D))


def kernel(pred, target, last):
    B, L, D = pred.shape
    ladj = ((last.astype(jnp.int32) + L - 1) % L).astype(jnp.int32)

    grid_spec = pltpu.PrefetchScalarGridSpec(
        num_scalar_prefetch=1,
        grid=(B,),
        in_specs=[
            pl.BlockSpec((1, _W, D), lambda b, ladj: (b, ladj[b] // _W, 0)),
            pl.BlockSpec((1, _W, D), lambda b, ladj: (b, ladj[b] // _W, 0)),
        ],
        out_specs=pl.BlockSpec(
            (1, 1), lambda b, ladj: (0, 0), memory_space=pltpu.SMEM
        ),
        scratch_shapes=[pltpu.VMEM((_W, D), jnp.float32)],
    )
    out = pl.pallas_call(
        _body,
        grid_spec=grid_spec,
        out_shape=jax.ShapeDtypeStruct((1, 1), jnp.float32),
        compiler_params=pltpu.CompilerParams(
            dimension_semantics=("arbitrary",),
        ),
    )(ladj, pred, target)
    return out[0, 0]


# speedup vs baseline: 1.6648x; 1.6648x over previous
import jax
import jax.numpy as jnp
from jax import lax
from jax.experimental import pallas as pl
from jax.experimental.pallas import tpu as pltpu

_B, _L, _D = 1024, 50, 1000
_W = 8
_K = 16


def _body(ladj_ref, *refs):
    pred_refs = refs[:_K]
    tgt_refs = refs[_K:2 * _K]
    out_ref = refs[2 * _K]
    acc_ref = refs[2 * _K + 1]
    g = pl.program_id(0)

    @pl.when(g == 0)
    def _init():
        acc_ref[...] = jnp.zeros_like(acc_ref)

    total = None
    for k in range(_K):
        sub = ladj_ref[g * _K + k] % _W
        p = pred_refs[k][0]
        t = tgt_refs[k][0].astype(jnp.float32)
        d = p - t
        rowmask = lax.broadcasted_iota(jnp.int32, (_W, _D), 0) == sub
        c = jnp.where(rowmask, d * d, 0.0)
        total = c if total is None else total + c
    acc_ref[...] += total

    @pl.when(g == _B // _K - 1)
    def _fin():
        out_ref[0, 0] = jnp.sum(acc_ref[...]) * (1.0 / (_B * _D))


def kernel(pred, target, last):
    B, L, D = pred.shape
    ladj = ((last.astype(jnp.int32) + L - 1) % L).astype(jnp.int32)

    def mk_spec(k):
        return pl.BlockSpec(
            (1, _W, D), lambda g, ladj, k=k: (g * _K + k, ladj[g * _K + k] // _W, 0)
        )

    grid_spec = pltpu.PrefetchScalarGridSpec(
        num_scalar_prefetch=1,
        grid=(B // _K,),
        in_specs=[mk_spec(k) for k in range(_K)] * 2,
        out_specs=pl.BlockSpec(
            (1, 1), lambda g, ladj: (0, 0), memory_space=pltpu.SMEM
        ),
        scratch_shapes=[pltpu.VMEM((_W, D), jnp.float32)],
    )
    out = pl.pallas_call(
        _body,
        grid_spec=grid_spec,
        out_shape=jax.ShapeDtypeStruct((1, 1), jnp.float32),
        compiler_params=pltpu.CompilerParams(
            dimension_semantics=("arbitrary",),
        ),
    )(ladj, *([pred] * _K), *([target] * _K))
    return out[0, 0]
